# Initial kernel scaffold; baseline (speedup 1.0000x reference)
import jax
import jax.numpy as jnp
from jax.experimental import pallas as pl


def kernel(x, edge_index, edge_attr, W, b, ln_gamma, ln_beta):
    raise NotImplementedError("write your pallas kernel here")


# speedup vs baseline: 10.6679x; 10.6679x over previous
import functools

import jax
import jax.numpy as jnp
from jax import lax
from jax.experimental import pallas as pl
from jax.experimental.pallas import tpu as pltpu
from jax.experimental.pallas import tpu_sc as plsc

N = 10000
NP = 10240
T = 12
C = 128
E = 320000
G = 128
EP = 327680
NG_TOT = EP // G
NTILES = 16
NG = NG_TOT // NTILES
RP = NP // NTILES
T_HALF = T // 2
BN = 640
EPS = 1e-5


def _sc_aggregate_body(xflat, src2, dst2, ew2, zout, s_out, dinv_out,
                       isrc_a, isrc_b, idst_a, idst_b, ival_a, ival_b,
                       dtmp_a, dtmp_b, dslice, ones_v, rows_a, rows_b,
                       degs, zacc, gsem_a, gsem_b, isem_a, isem_b):
    cc = lax.axis_index("c")
    ss = lax.axis_index("s")
    rbase = ss * RP
    gbase = ss * NG

    def vec_scale_rows(buf, coef16s_ref, coef_off, square):
        def chunk(kk, carry):
            c16 = coef16s_ref[pl.ds(coef_off + kk * 16, 16)]
            for j in range(16):
                cv = c16[j]
                if square:
                    cv = cv * cv
                r = kk * 16 + j
                for k in range(8):
                    sl = pl.ds(k * 16, 16)
                    buf[r, sl] = buf[r, sl] * cv
            return carry
        lax.fori_loop(0, G // 16, chunk, 0)

    def fill_ones(i, carry):
        ones_v[pl.ds(i * 16, 16)] = jnp.full((16,), 1.0, jnp.float32)
        return carry
    lax.fori_loop(0, RP // 16, fill_ones, 0)
    pltpu.sync_copy(ones_v, degs.at[pl.ds(rbase, RP)])
    plsc.subcore_barrier()

    def deg_body(g, carry):
        grow = gbase + g
        pltpu.sync_copy(dst2.at[grow], idst_a)
        pltpu.sync_copy(ew2.at[grow], ival_a)
        pltpu.sync_copy(ival_a, degs.at[idst_a], add=True)
        return carry
    lax.fori_loop(0, NG, deg_body, 0)
    plsc.subcore_barrier()

    pltpu.sync_copy(degs.at[pl.ds(rbase, RP)], dslice)

    def rsqrt_body(i, carry):
        sl = pl.ds(i * 16, 16)
        v = dslice[sl]
        iv = lax.bitcast_convert_type(v, jnp.int32)
        y = lax.bitcast_convert_type(jnp.int32(0x5F3759DF) - (iv >> 1),
                                     jnp.float32)
        for _ in range(4):
            y = y * (1.5 - 0.5 * v * y * y)
        dslice[sl] = y
        return carry
    lax.fori_loop(0, RP // 16, rsqrt_body, 0)
    pltpu.sync_copy(dslice, dinv_out.at[pl.ds(rbase, RP)])
    plsc.subcore_barrier()

    def s_body(g, carry):
        grow = gbase + g
        pltpu.sync_copy(src2.at[grow], isrc_a)
        pltpu.sync_copy(dst2.at[grow], idst_a)
        pltpu.sync_copy(ew2.at[grow], ival_a)
        pltpu.async_copy(dinv_out.at[isrc_a], dtmp_a, gsem_a).wait()
        pltpu.async_copy(dinv_out.at[idst_a], dtmp_b, gsem_b).wait()
        for k in range(8):
            sl = pl.ds(k * 16, 16)
            ival_a[sl] = ival_a[sl] * dtmp_a[sl] * dtmp_b[sl]
        pltpu.sync_copy(ival_a, s_out.at[grow])
        return carry
    lax.fori_loop(0, NG, s_body, 0)

    ibufs = ((isrc_a, idst_a, ival_a, isem_a), (isrc_b, idst_b, ival_b,
                                                isem_b))
    rbufs = ((rows_a, gsem_a), (rows_b, gsem_b))

    def idx_copies(g, bsel):
        isrc, idst, ival, isem = ibufs[bsel]
        grow = gbase + g
        return (pltpu.make_async_copy(src2.at[grow], isrc, isem),
                pltpu.make_async_copy(dst2.at[grow], idst, isem),
                pltpu.make_async_copy(s_out.at[grow], ival, isem))

    def idx_start(g, bsel):
        for cp in idx_copies(g, bsel):
            cp.start()

    def idx_wait(g, bsel):
        for cp in idx_copies(g, bsel):
            cp.wait()

    def add_off(bsel, off):
        isrc = ibufs[bsel][0]
        for k in range(8):
            sl = pl.ds(k * 16, 16)
            isrc[sl] = isrc[sl] + off

    def gather_start(bsel):
        buf, gsem = rbufs[bsel]
        pltpu.make_async_copy(xflat.at[ibufs[bsel][0]], buf, gsem).start()

    def gather_wait(bsel):
        buf, gsem = rbufs[bsel]
        pltpu.make_async_copy(xflat.at[ibufs[bsel][0]], buf, gsem).wait()

    t0 = cc * T_HALF

    def chunk_body(ci, carry):
        t = t0 + ci
        toff = t * NP

        def init_piece(p, carry2):
            row0 = rbase + p * G
            pltpu.sync_copy(xflat.at[pl.ds(toff + row0, G)], rows_a)
            vec_scale_rows(rows_a, dslice, p * G, square=True)
            pltpu.sync_copy(rows_a, zacc.at[pl.ds(row0, G)])
            return carry2
        lax.fori_loop(0, RP // G, init_piece, 0)
        plsc.subcore_barrier()

        idx_start(0, 0)
        idx_start(1, 1)
        idx_wait(0, 0)
        add_off(0, toff)
        gather_start(0)

        def group_body(i, carry):
            for b in range(2):
                g = i * 2 + b
                nb = 1 - b

                @pl.when(g + 1 < NG)
                def _():
                    idx_wait(g + 1, nb)
                    add_off(nb, toff)
                    gather_start(nb)

                gather_wait(b)
                vec_scale_rows(rbufs[b][0], ibufs[b][2], 0, square=False)
                pltpu.sync_copy(rbufs[b][0], zacc.at[ibufs[b][1]],
                                add=True)

                @pl.when(g + 2 < NG)
                def _():
                    idx_start(g + 2, b)
            return carry
        lax.fori_loop(0, NG // 2, group_body, 0)
        plsc.subcore_barrier()

        def out_piece(p, carry2):
            row0 = rbase + p * G
            pltpu.sync_copy(zacc.at[pl.ds(row0, G)], rows_a)
            pltpu.sync_copy(rows_a, zout.at[t, pl.ds(row0, G)])
            return carry2
        lax.fori_loop(0, RP // G, out_piece, 0)
        return carry

    lax.fori_loop(0, T_HALF, chunk_body, 0)


_sc_aggregate = functools.partial(
    pl.kernel,
    out_type=(
        jax.ShapeDtypeStruct((T, NP, C), jnp.float32),
        jax.ShapeDtypeStruct((NG_TOT, G), jnp.float32),
        jax.ShapeDtypeStruct((NP,), jnp.float32),
    ),
    mesh=plsc.VectorSubcoreMesh(core_axis_name="c", subcore_axis_name="s"),
    compiler_params=pltpu.CompilerParams(needs_layout_passes=False),
    scratch_types=[
        pltpu.VMEM((G,), jnp.int32),
        pltpu.VMEM((G,), jnp.int32),
        pltpu.VMEM((G,), jnp.int32),
        pltpu.VMEM((G,), jnp.int32),
        pltpu.VMEM((G,), jnp.float32),
        pltpu.VMEM((G,), jnp.float32),
        pltpu.VMEM((G,), jnp.float32),
        pltpu.VMEM((G,), jnp.float32),
        pltpu.VMEM((RP,), jnp.float32),
        pltpu.VMEM((RP,), jnp.float32),
        pltpu.VMEM((G, C), jnp.float32),
        pltpu.VMEM((G, C), jnp.float32),
        pltpu.VMEM_SHARED((NP,), jnp.float32),
        pltpu.VMEM_SHARED((NP, C), jnp.float32),
        pltpu.SemaphoreType.DMA,
        pltpu.SemaphoreType.DMA,
        pltpu.SemaphoreType.DMA,
        pltpu.SemaphoreType.DMA,
    ],
)(_sc_aggregate_body)


def _mm_body(z_ref, w_ref, b_ref, y_ref, s_ref, s2_ref):
    t = pl.program_id(0)
    nb = pl.program_id(1)
    z = z_ref[0]
    y = jnp.dot(z, w_ref[...], preferred_element_type=jnp.float32)
    y = jnp.maximum(y + b_ref[...], 0.0)
    rid = lax.broadcasted_iota(jnp.int32, (BN, 1), 0) + nb * BN
    y = jnp.where(rid < N, y, 0.0)
    y_ref[0] = y

    @pl.when((t == 0) & (nb == 0))
    def _():
        s_ref[...] = jnp.zeros_like(s_ref)
        s2_ref[...] = jnp.zeros_like(s2_ref)
    s_ref[0:1, :] = s_ref[0:1, :] + jnp.sum(y, axis=0, keepdims=True)
    s2_ref[0:1, :] = s2_ref[0:1, :] + jnp.sum(y * y, axis=0, keepdims=True)


_mm_call = pl.pallas_call(
    _mm_body,
    grid=(T, NP // BN),
    in_specs=[
        pl.BlockSpec((1, BN, C), lambda t, nb: (t, nb, 0)),
        pl.BlockSpec((C, C), lambda t, nb: (0, 0)),
        pl.BlockSpec((1, C), lambda t, nb: (0, 0)),
    ],
    out_specs=[
        pl.BlockSpec((1, BN, C), lambda t, nb: (t, nb, 0)),
        pl.BlockSpec((8, C), lambda t, nb: (0, 0)),
        pl.BlockSpec((8, C), lambda t, nb: (0, 0)),
    ],
    out_shape=[
        jax.ShapeDtypeStruct((T, NP, C), jnp.float32),
        jax.ShapeDtypeStruct((8, C), jnp.float32),
        jax.ShapeDtypeStruct((8, C), jnp.float32),
    ],
)


def _ln_body(y_ref, s_ref, s2_ref, g_ref, be_ref, o_ref):
    inv = 1.0 / float(N * T)
    mu = s_ref[0:1, :] * inv
    var = s2_ref[0:1, :] * inv - mu * mu
    rstd = lax.rsqrt(var + EPS)
    for t in range(T):
        zt = (y_ref[t] - mu) * rstd
        o_ref[t] = zt * g_ref[:, t:t + 1] + be_ref[:, t:t + 1]


_ln_call = pl.pallas_call(
    _ln_body,
    grid=(NP // BN,),
    in_specs=[
        pl.BlockSpec((T, BN, C), lambda nb: (0, nb, 0)),
        pl.BlockSpec((8, C), lambda nb: (0, 0)),
        pl.BlockSpec((8, C), lambda nb: (0, 0)),
        pl.BlockSpec((BN, T), lambda nb: (nb, 0)),
        pl.BlockSpec((BN, T), lambda nb: (nb, 0)),
    ],
    out_specs=pl.BlockSpec((T, BN, C), lambda nb: (0, nb, 0)),
    out_shape=jax.ShapeDtypeStruct((T, NP, C), jnp.float32),
)


def kernel(x, edge_index, edge_attr, W, b, ln_gamma, ln_beta):
    src = edge_index[0]
    dst = edge_index[1]
    pad = EP - E
    fill = (jnp.arange(pad, dtype=jnp.int32) * 37) % N
    src2 = jnp.concatenate([src, fill]).reshape(NG_TOT, G)
    dst2 = jnp.concatenate([dst, fill]).reshape(NG_TOT, G)
    ew2 = jnp.concatenate(
        [edge_attr, jnp.zeros((pad,), jnp.float32)]).reshape(NG_TOT, G)

    xT = jnp.transpose(x, (2, 0, 1))
    xflat = jnp.pad(xT, ((0, 0), (0, NP - N), (0, 0))).reshape(T * NP, C)

    zout, _, _ = _sc_aggregate(xflat, src2, dst2, ew2)

    y, s, s2 = _mm_call(zout, W, b.reshape(1, C))
    gp = jnp.pad(ln_gamma, ((0, NP - N), (0, 0)))
    bp = jnp.pad(ln_beta, ((0, NP - N), (0, 0)))
    outT = _ln_call(y, s, s2, gp, bp)

    return jnp.transpose(outT[:, :N, :], (1, 2, 0))


# speedup vs baseline: 14.1532x; 1.3267x over previous
import functools

import jax
import jax.numpy as jnp
from jax import lax
from jax.experimental import pallas as pl
from jax.experimental.pallas import tpu as pltpu
from jax.experimental.pallas import tpu_sc as plsc

N = 10000
NP = 10240
T = 12
C = 128
E = 320000
G = 128
EP = 327680
NG_TOT = EP // G
NTILES = 16
NG = NG_TOT // NTILES
RP = NP // NTILES
T_HALF = T // 2
BN = 640
EPS = 1e-5


def _sc_aggregate_body(xflat, src2, dst2, ew2, zout, s_out, dinv_out,
                       isrc_a, isrc_b, idst_a, idst_b, ival_a, ival_b,
                       dtmp_a, dtmp_b, sdst_a, sdst_b, dslice, ones_v,
                       rows_a, rows_b, degs, zacc,
                       gsem_a, gsem_b, isem_a, isem_b, ssem_a, ssem_b):
    cc = lax.axis_index("c")
    ss = lax.axis_index("s")
    rbase = ss * RP
    gbase = ss * NG

    def vec_scale_rows(buf, coef16s_ref, coef_off, square):
        def chunk(kk, carry):
            c16 = coef16s_ref[pl.ds(coef_off + kk * 16, 16)]
            for j in range(16):
                cv = c16[j]
                if square:
                    cv = cv * cv
                r = kk * 16 + j
                for k in range(8):
                    sl = pl.ds(k * 16, 16)
                    buf[r, sl] = buf[r, sl] * cv
            return carry
        lax.fori_loop(0, G // 16, chunk, 0)

    def fill_ones(i, carry):
        ones_v[pl.ds(i * 16, 16)] = jnp.full((16,), 1.0, jnp.float32)
        return carry
    lax.fori_loop(0, RP // 16, fill_ones, 0)
    pltpu.sync_copy(ones_v, degs.at[pl.ds(rbase, RP)])
    plsc.subcore_barrier()

    def deg_body(g, carry):
        grow = gbase + g
        ca = pltpu.make_async_copy(dst2.at[grow], idst_a, isem_a)
        cb = pltpu.make_async_copy(ew2.at[grow], ival_a, isem_b)
        ca.start()
        cb.start()
        ca.wait()
        cb.wait()
        pltpu.sync_copy(ival_a, degs.at[idst_a], add=True)
        return carry
    lax.fori_loop(0, NG, deg_body, 0)
    plsc.subcore_barrier()

    pltpu.sync_copy(degs.at[pl.ds(rbase, RP)], dslice)

    def rsqrt_body(i, carry):
        sl = pl.ds(i * 16, 16)
        v = dslice[sl]
        iv = lax.bitcast_convert_type(v, jnp.int32)
        y = lax.bitcast_convert_type(jnp.int32(0x5F3759DF) - (iv >> 1),
                                     jnp.float32)
        for _ in range(4):
            y = y * (1.5 - 0.5 * v * y * y)
        dslice[sl] = y
        return carry
    lax.fori_loop(0, RP // 16, rsqrt_body, 0)
    pltpu.sync_copy(dslice, dinv_out.at[pl.ds(rbase, RP)])
    plsc.subcore_barrier()

    def s_body(g, carry):
        grow = gbase + g
        ca = pltpu.make_async_copy(src2.at[grow], isrc_a, isem_a)
        cb = pltpu.make_async_copy(dst2.at[grow], idst_a, isem_b)
        cc2 = pltpu.make_async_copy(ew2.at[grow], ival_a, gsem_a)
        ca.start()
        cb.start()
        cc2.start()
        ca.wait()
        cb.wait()
        cc2.wait()
        ga = pltpu.async_copy(dinv_out.at[isrc_a], dtmp_a, gsem_a)
        gb = pltpu.async_copy(dinv_out.at[idst_a], dtmp_b, gsem_b)
        ga.wait()
        gb.wait()
        for k in range(8):
            sl = pl.ds(k * 16, 16)
            ival_a[sl] = ival_a[sl] * dtmp_a[sl] * dtmp_b[sl]
        pltpu.sync_copy(ival_a, s_out.at[grow])
        return carry
    lax.fori_loop(0, NG, s_body, 0)

    ibufs = ((isrc_a, idst_a, ival_a, isem_a), (isrc_b, idst_b, ival_b,
                                                isem_b))
    rbufs = ((rows_a, gsem_a), (rows_b, gsem_b))
    sbufs = ((sdst_a, ssem_a), (sdst_b, ssem_b))

    def idx_copies(g, bsel):
        isrc, idst, ival, isem = ibufs[bsel]
        grow = gbase + g
        return (pltpu.make_async_copy(src2.at[grow], isrc, isem),
                pltpu.make_async_copy(dst2.at[grow], idst, isem),
                pltpu.make_async_copy(s_out.at[grow], ival, isem))

    def idx_start(g, bsel):
        for cp in idx_copies(g, bsel):
            cp.start()

    def idx_wait(g, bsel):
        for cp in idx_copies(g, bsel):
            cp.wait()

    def add_off(bsel, off):
        isrc = ibufs[bsel][0]
        for k in range(8):
            sl = pl.ds(k * 16, 16)
            isrc[sl] = isrc[sl] + off

    def gather_start(bsel):
        buf, gsem = rbufs[bsel]
        pltpu.make_async_copy(xflat.at[ibufs[bsel][0]], buf, gsem).start()

    def gather_wait(bsel):
        buf, gsem = rbufs[bsel]
        pltpu.make_async_copy(xflat.at[ibufs[bsel][0]], buf, gsem).wait()

    def scatter_start(bsel):
        idst = ibufs[bsel][1]
        sdst, ssem = sbufs[bsel]
        for k in range(8):
            sl = pl.ds(k * 16, 16)
            sdst[sl] = idst[sl]
        pltpu.async_copy(rbufs[bsel][0], zacc.at[sdst], ssem, add=True)

    def scatter_wait(bsel):
        sdst, ssem = sbufs[bsel]
        pltpu.make_async_copy(rbufs[bsel][0], zacc.at[sdst], ssem).wait()

    t0 = cc * T_HALF

    def chunk_body(ci, carry):
        t = t0 + ci
        toff = t * NP

        def init_piece(p, carry2):
            row0 = rbase + p * G
            pltpu.sync_copy(xflat.at[pl.ds(toff + row0, G)], rows_a)
            vec_scale_rows(rows_a, dslice, p * G, square=True)
            pltpu.sync_copy(rows_a, zacc.at[pl.ds(row0, G)])
            return carry2
        lax.fori_loop(0, RP // G, init_piece, 0)
        plsc.subcore_barrier()

        idx_start(0, 0)
        idx_start(1, 1)
        idx_wait(0, 0)
        add_off(0, toff)
        gather_start(0)

        def group_body(i, carry):
            for b in range(2):
                g = i * 2 + b
                nb = 1 - b

                @pl.when(g + 1 < NG)
                def _():
                    idx_wait(g + 1, nb)
                    add_off(nb, toff)

                    @pl.when(g >= 1)
                    def _():
                        scatter_wait(nb)
                    gather_start(nb)

                gather_wait(b)
                vec_scale_rows(rbufs[b][0], ibufs[b][2], 0, square=False)
                scatter_start(b)

                @pl.when(g + 2 < NG)
                def _():
                    idx_start(g + 2, b)
            return carry
        lax.fori_loop(0, NG // 2, group_body, 0)
        scatter_wait(0)
        scatter_wait(1)
        plsc.subcore_barrier()

        def out_piece(p, carry2):
            row0 = rbase + p * G
            pltpu.sync_copy(zacc.at[pl.ds(row0, G)], rows_a)
            pltpu.sync_copy(rows_a, zout.at[t, pl.ds(row0, G)])
            return carry2
        lax.fori_loop(0, RP // G, out_piece, 0)
        return carry

    lax.fori_loop(0, T_HALF, chunk_body, 0)


_sc_aggregate = functools.partial(
    pl.kernel,
    out_type=(
        jax.ShapeDtypeStruct((T, NP, C), jnp.float32),
        jax.ShapeDtypeStruct((NG_TOT, G), jnp.float32),
        jax.ShapeDtypeStruct((NP,), jnp.float32),
    ),
    mesh=plsc.VectorSubcoreMesh(core_axis_name="c", subcore_axis_name="s"),
    compiler_params=pltpu.CompilerParams(needs_layout_passes=False),
    scratch_types=[
        pltpu.VMEM((G,), jnp.int32),
        pltpu.VMEM((G,), jnp.int32),
        pltpu.VMEM((G,), jnp.int32),
        pltpu.VMEM((G,), jnp.int32),
        pltpu.VMEM((G,), jnp.float32),
        pltpu.VMEM((G,), jnp.float32),
        pltpu.VMEM((G,), jnp.float32),
        pltpu.VMEM((G,), jnp.float32),
        pltpu.VMEM((G,), jnp.int32),
        pltpu.VMEM((G,), jnp.int32),
        pltpu.VMEM((RP,), jnp.float32),
        pltpu.VMEM((RP,), jnp.float32),
        pltpu.VMEM((G, C), jnp.float32),
        pltpu.VMEM((G, C), jnp.float32),
        pltpu.VMEM_SHARED((NP,), jnp.float32),
        pltpu.VMEM_SHARED((NP, C), jnp.float32),
        pltpu.SemaphoreType.DMA,
        pltpu.SemaphoreType.DMA,
        pltpu.SemaphoreType.DMA,
        pltpu.SemaphoreType.DMA,
        pltpu.SemaphoreType.DMA,
        pltpu.SemaphoreType.DMA,
    ],
)(_sc_aggregate_body)


def _mm_body(z_ref, w_ref, b_ref, y_ref, s_ref, s2_ref):
    t = pl.program_id(0)
    nb = pl.program_id(1)
    z = z_ref[0]
    y = jnp.dot(z, w_ref[...], preferred_element_type=jnp.float32)
    y = jnp.maximum(y + b_ref[...], 0.0)
    rid = lax.broadcasted_iota(jnp.int32, (BN, 1), 0) + nb * BN
    y = jnp.where(rid < N, y, 0.0)
    y_ref[0] = y

    @pl.when((t == 0) & (nb == 0))
    def _():
        s_ref[...] = jnp.zeros_like(s_ref)
        s2_ref[...] = jnp.zeros_like(s2_ref)
    s_ref[0:1, :] = s_ref[0:1, :] + jnp.sum(y, axis=0, keepdims=True)
    s2_ref[0:1, :] = s2_ref[0:1, :] + jnp.sum(y * y, axis=0, keepdims=True)


_mm_call = pl.pallas_call(
    _mm_body,
    grid=(T, NP // BN),
    in_specs=[
        pl.BlockSpec((1, BN, C), lambda t, nb: (t, nb, 0)),
        pl.BlockSpec((C, C), lambda t, nb: (0, 0)),
        pl.BlockSpec((1, C), lambda t, nb: (0, 0)),
    ],
    out_specs=[
        pl.BlockSpec((1, BN, C), lambda t, nb: (t, nb, 0)),
        pl.BlockSpec((8, C), lambda t, nb: (0, 0)),
        pl.BlockSpec((8, C), lambda t, nb: (0, 0)),
    ],
    out_shape=[
        jax.ShapeDtypeStruct((T, NP, C), jnp.float32),
        jax.ShapeDtypeStruct((8, C), jnp.float32),
        jax.ShapeDtypeStruct((8, C), jnp.float32),
    ],
)


def _ln_body(y_ref, s_ref, s2_ref, g_ref, be_ref, o_ref):
    inv = 1.0 / float(N * T)
    mu = s_ref[0:1, :] * inv
    var = s2_ref[0:1, :] * inv - mu * mu
    rstd = lax.rsqrt(var + EPS)
    for t in range(T):
        zt = (y_ref[t] - mu) * rstd
        o_ref[t] = zt * g_ref[:, t:t + 1] + be_ref[:, t:t + 1]


_ln_call = pl.pallas_call(
    _ln_body,
    grid=(NP // BN,),
    in_specs=[
        pl.BlockSpec((T, BN, C), lambda nb: (0, nb, 0)),
        pl.BlockSpec((8, C), lambda nb: (0, 0)),
        pl.BlockSpec((8, C), lambda nb: (0, 0)),
        pl.BlockSpec((BN, T), lambda nb: (nb, 0)),
        pl.BlockSpec((BN, T), lambda nb: (nb, 0)),
    ],
    out_specs=pl.BlockSpec((T, BN, C), lambda nb: (0, nb, 0)),
    out_shape=jax.ShapeDtypeStruct((T, NP, C), jnp.float32),
)


def kernel(x, edge_index, edge_attr, W, b, ln_gamma, ln_beta):
    src = edge_index[0]
    dst = edge_index[1]
    pad = EP - E
    fill = (jnp.arange(pad, dtype=jnp.int32) * 37) % N
    src2 = jnp.concatenate([src, fill]).reshape(NG_TOT, G)
    dst2 = jnp.concatenate([dst, fill]).reshape(NG_TOT, G)
    ew2 = jnp.concatenate(
        [edge_attr, jnp.zeros((pad,), jnp.float32)]).reshape(NG_TOT, G)

    xT = jnp.transpose(x, (2, 0, 1))
    xflat = jnp.pad(xT, ((0, 0), (0, NP - N), (0, 0))).reshape(T * NP, C)

    zout, _, _ = _sc_aggregate(xflat, src2, dst2, ew2)

    y, s, s2 = _mm_call(zout, W, b.reshape(1, C))
    gp = jnp.pad(ln_gamma, ((0, NP - N), (0, 0)))
    bp = jnp.pad(ln_beta, ((0, NP - N), (0, 0)))
    outT = _ln_call(y, s, s2, gp, bp)

    return jnp.transpose(outT[:, :N, :], (1, 2, 0))


# speedup vs baseline: 14.5221x; 1.0261x over previous
import functools

import jax
import jax.numpy as jnp
from jax import lax
from jax.experimental import pallas as pl
from jax.experimental.pallas import tpu as pltpu
from jax.experimental.pallas import tpu_sc as plsc

N = 10000
NP = 10240
T = 12
C = 128
E = 320000
G = 128
EP = 327680
NG_TOT = EP // G
NTILES = 16
NG = NG_TOT // NTILES
RP = NP // NTILES
T_HALF = T // 2
BN = 640
EPS = 1e-5


def _sc_aggregate_body(xflat, src2, dst2, ew2, zout, s_out, dinv_out,
                       isrc_a, isrc_b, idst_a, idst_b, ival_a, ival_b,
                       dtmp_a, dtmp_b, sdst_a, sdst_b, dslice, ones_v,
                       rows_a, rows_b, degs, zacc,
                       gsem_a, gsem_b, isem_a, isem_b, ssem_a, ssem_b):
    cc = lax.axis_index("c")
    ss = lax.axis_index("s")
    rbase = ss * RP
    gbase = ss * NG

    def vec_scale_rows(buf, coef16s_ref, coef_off, square):
        def chunk(kk, carry):
            c16 = coef16s_ref[pl.ds(coef_off + kk * 16, 16)]
            for j in range(16):
                cv = c16[j]
                if square:
                    cv = cv * cv
                r = kk * 16 + j
                for k in range(8):
                    sl = pl.ds(k * 16, 16)
                    buf[r, sl] = buf[r, sl] * cv
            return carry
        lax.fori_loop(0, G // 16, chunk, 0)

    def fill_ones(i, carry):
        ones_v[pl.ds(i * 16, 16)] = jnp.full((16,), 1.0, jnp.float32)
        return carry
    lax.fori_loop(0, RP // 16, fill_ones, 0)
    pltpu.sync_copy(ones_v, degs.at[pl.ds(rbase, RP)])
    plsc.subcore_barrier()

    dbun = ((idst_a, ival_a, isem_a, ssem_a), (idst_b, ival_b, isem_b,
                                               ssem_b))

    def deg_loads(g, b):
        idst, ival, isem, _ = dbun[b]
        grow = gbase + g
        return (pltpu.make_async_copy(dst2.at[grow], idst, isem),
                pltpu.make_async_copy(ew2.at[grow], ival, isem))

    def deg_scat_wait(b):
        idst, ival, _, ssem = dbun[b]
        pltpu.make_async_copy(ival, degs.at[idst], ssem).wait()

    for cp in deg_loads(0, 0):
        cp.start()

    def deg_body(i, carry):
        for b in range(2):
            g = i * 2 + b
            nb = 1 - b
            for cp in deg_loads(g, b):
                cp.wait()

            @pl.when(g + 1 < NG)
            def _():
                @pl.when(g >= 1)
                def _():
                    deg_scat_wait(nb)
                for cp in deg_loads(g + 1, nb):
                    cp.start()

            idst, ival, _, ssem = dbun[b]
            pltpu.async_copy(ival, degs.at[idst], ssem, add=True)
        return carry
    lax.fori_loop(0, NG // 2, deg_body, 0)
    deg_scat_wait(0)
    deg_scat_wait(1)
    plsc.subcore_barrier()

    pltpu.sync_copy(degs.at[pl.ds(rbase, RP)], dslice)

    def rsqrt_body(i, carry):
        sl = pl.ds(i * 16, 16)
        v = dslice[sl]
        iv = lax.bitcast_convert_type(v, jnp.int32)
        y = lax.bitcast_convert_type(jnp.int32(0x5F3759DF) - (iv >> 1),
                                     jnp.float32)
        for _ in range(4):
            y = y * (1.5 - 0.5 * v * y * y)
        dslice[sl] = y
        return carry
    lax.fori_loop(0, RP // 16, rsqrt_body, 0)
    pltpu.sync_copy(dslice, dinv_out.at[pl.ds(rbase, RP)])
    plsc.subcore_barrier()

    sbun = ((isrc_a, idst_a, ival_a, isem_a, ssem_a),
            (isrc_b, idst_b, ival_b, isem_b, ssem_b))

    def s_loads(g, b):
        isrc, idst, ival, isem, _ = sbun[b]
        grow = gbase + g
        return (pltpu.make_async_copy(src2.at[grow], isrc, isem),
                pltpu.make_async_copy(dst2.at[grow], idst, isem),
                pltpu.make_async_copy(ew2.at[grow], ival, isem))

    def s_store_wait(g, b):
        ival, ssem = sbun[b][2], sbun[b][4]
        pltpu.make_async_copy(ival, s_out.at[gbase + g], ssem).wait()

    for cp in s_loads(0, 0):
        cp.start()

    def s_body(i, carry):
        for b in range(2):
            g = i * 2 + b
            nb = 1 - b
            for cp in s_loads(g, b):
                cp.wait()

            @pl.when(g + 1 < NG)
            def _():
                @pl.when(g >= 1)
                def _():
                    s_store_wait(g - 1, nb)
                for cp in s_loads(g + 1, nb):
                    cp.start()

            isrc, idst, ival, isem, ssem = sbun[b]
            ga = pltpu.async_copy(dinv_out.at[isrc], dtmp_a, gsem_a)
            gb = pltpu.async_copy(dinv_out.at[idst], dtmp_b, gsem_b)
            ga.wait()
            gb.wait()
            for k in range(8):
                sl = pl.ds(k * 16, 16)
                ival[sl] = ival[sl] * dtmp_a[sl] * dtmp_b[sl]
            pltpu.async_copy(ival, s_out.at[gbase + g], ssem)
        return carry
    lax.fori_loop(0, NG // 2, s_body, 0)
    s_store_wait(NG - 2, 0)
    s_store_wait(NG - 1, 1)

    ibufs = ((isrc_a, idst_a, ival_a, isem_a), (isrc_b, idst_b, ival_b,
                                                isem_b))
    rbufs = ((rows_a, gsem_a), (rows_b, gsem_b))
    sbufs = ((sdst_a, ssem_a), (sdst_b, ssem_b))

    def idx_copies(g, bsel):
        isrc, idst, ival, isem = ibufs[bsel]
        grow = gbase + g
        return (pltpu.make_async_copy(src2.at[grow], isrc, isem),
                pltpu.make_async_copy(dst2.at[grow], idst, isem),
                pltpu.make_async_copy(s_out.at[grow], ival, isem))

    def idx_start(g, bsel):
        for cp in idx_copies(g, bsel):
            cp.start()

    def idx_wait(g, bsel):
        for cp in idx_copies(g, bsel):
            cp.wait()

    def add_off(bsel, off):
        isrc = ibufs[bsel][0]
        for k in range(8):
            sl = pl.ds(k * 16, 16)
            isrc[sl] = isrc[sl] + off

    def gather_start(bsel):
        buf, gsem = rbufs[bsel]
        pltpu.make_async_copy(xflat.at[ibufs[bsel][0]], buf, gsem).start()

    def gather_wait(bsel):
        buf, gsem = rbufs[bsel]
        pltpu.make_async_copy(xflat.at[ibufs[bsel][0]], buf, gsem).wait()

    def scatter_start(bsel):
        idst = ibufs[bsel][1]
        sdst, ssem = sbufs[bsel]
        for k in range(8):
            sl = pl.ds(k * 16, 16)
            sdst[sl] = idst[sl]
        pltpu.async_copy(rbufs[bsel][0], zacc.at[sdst], ssem, add=True)

    def scatter_wait(bsel):
        sdst, ssem = sbufs[bsel]
        pltpu.make_async_copy(rbufs[bsel][0], zacc.at[sdst], ssem).wait()

    t0 = cc * T_HALF

    def chunk_body(ci, carry):
        t = t0 + ci
        toff = t * NP

        def init_piece(p, carry2):
            row0 = rbase + p * G
            pltpu.sync_copy(xflat.at[pl.ds(toff + row0, G)], rows_a)
            vec_scale_rows(rows_a, dslice, p * G, square=True)
            pltpu.sync_copy(rows_a, zacc.at[pl.ds(row0, G)])
            return carry2
        lax.fori_loop(0, RP // G, init_piece, 0)
        plsc.subcore_barrier()

        idx_start(0, 0)
        idx_start(1, 1)
        idx_wait(0, 0)
        add_off(0, toff)
        gather_start(0)

        def group_body(i, carry):
            for b in range(2):
                g = i * 2 + b
                nb = 1 - b

                @pl.when(g + 1 < NG)
                def _():
                    idx_wait(g + 1, nb)
                    add_off(nb, toff)

                    @pl.when(g >= 1)
                    def _():
                        scatter_wait(nb)
                    gather_start(nb)

                gather_wait(b)
                vec_scale_rows(rbufs[b][0], ibufs[b][2], 0, square=False)
                scatter_start(b)

                @pl.when(g + 2 < NG)
                def _():
                    idx_start(g + 2, b)
            return carry
        lax.fori_loop(0, NG // 2, group_body, 0)
        scatter_wait(0)
        scatter_wait(1)
        plsc.subcore_barrier()

        def out_piece(p, carry2):
            row0 = rbase + p * G
            pltpu.sync_copy(zacc.at[pl.ds(row0, G)], rows_a)
            pltpu.sync_copy(rows_a, zout.at[t, pl.ds(row0, G)])
            return carry2
        lax.fori_loop(0, RP // G, out_piece, 0)
        return carry

    lax.fori_loop(0, T_HALF, chunk_body, 0)


_sc_aggregate = functools.partial(
    pl.kernel,
    out_type=(
        jax.ShapeDtypeStruct((T, NP, C), jnp.float32),
        jax.ShapeDtypeStruct((NG_TOT, G), jnp.float32),
        jax.ShapeDtypeStruct((NP,), jnp.float32),
    ),
    mesh=plsc.VectorSubcoreMesh(core_axis_name="c", subcore_axis_name="s"),
    compiler_params=pltpu.CompilerParams(needs_layout_passes=False),
    scratch_types=[
        pltpu.VMEM((G,), jnp.int32),
        pltpu.VMEM((G,), jnp.int32),
        pltpu.VMEM((G,), jnp.int32),
        pltpu.VMEM((G,), jnp.int32),
        pltpu.VMEM((G,), jnp.float32),
        pltpu.VMEM((G,), jnp.float32),
        pltpu.VMEM((G,), jnp.float32),
        pltpu.VMEM((G,), jnp.float32),
        pltpu.VMEM((G,), jnp.int32),
        pltpu.VMEM((G,), jnp.int32),
        pltpu.VMEM((RP,), jnp.float32),
        pltpu.VMEM((RP,), jnp.float32),
        pltpu.VMEM((G, C), jnp.float32),
        pltpu.VMEM((G, C), jnp.float32),
        pltpu.VMEM_SHARED((NP,), jnp.float32),
        pltpu.VMEM_SHARED((NP, C), jnp.float32),
        pltpu.SemaphoreType.DMA,
        pltpu.SemaphoreType.DMA,
        pltpu.SemaphoreType.DMA,
        pltpu.SemaphoreType.DMA,
        pltpu.SemaphoreType.DMA,
        pltpu.SemaphoreType.DMA,
    ],
)(_sc_aggregate_body)


def _mm_body(z_ref, w_ref, b_ref, y_ref, s_ref, s2_ref):
    t = pl.program_id(0)
    nb = pl.program_id(1)
    z = z_ref[0]
    y = jnp.dot(z, w_ref[...], preferred_element_type=jnp.float32)
    y = jnp.maximum(y + b_ref[...], 0.0)
    rid = lax.broadcasted_iota(jnp.int32, (BN, 1), 0) + nb * BN
    y = jnp.where(rid < N, y, 0.0)
    y_ref[0] = y

    @pl.when((t == 0) & (nb == 0))
    def _():
        s_ref[...] = jnp.zeros_like(s_ref)
        s2_ref[...] = jnp.zeros_like(s2_ref)
    s_ref[0:1, :] = s_ref[0:1, :] + jnp.sum(y, axis=0, keepdims=True)
    s2_ref[0:1, :] = s2_ref[0:1, :] + jnp.sum(y * y, axis=0, keepdims=True)


_mm_call = pl.pallas_call(
    _mm_body,
    grid=(T, NP // BN),
    in_specs=[
        pl.BlockSpec((1, BN, C), lambda t, nb: (t, nb, 0)),
        pl.BlockSpec((C, C), lambda t, nb: (0, 0)),
        pl.BlockSpec((1, C), lambda t, nb: (0, 0)),
    ],
    out_specs=[
        pl.BlockSpec((1, BN, C), lambda t, nb: (t, nb, 0)),
        pl.BlockSpec((8, C), lambda t, nb: (0, 0)),
        pl.BlockSpec((8, C), lambda t, nb: (0, 0)),
    ],
    out_shape=[
        jax.ShapeDtypeStruct((T, NP, C), jnp.float32),
        jax.ShapeDtypeStruct((8, C), jnp.float32),
        jax.ShapeDtypeStruct((8, C), jnp.float32),
    ],
)


def _ln_body(y_ref, s_ref, s2_ref, g_ref, be_ref, o_ref):
    inv = 1.0 / float(N * T)
    mu = s_ref[0:1, :] * inv
    var = s2_ref[0:1, :] * inv - mu * mu
    rstd = lax.rsqrt(var + EPS)
    for t in range(T):
        zt = (y_ref[t] - mu) * rstd
        o_ref[t] = zt * g_ref[:, t:t + 1] + be_ref[:, t:t + 1]


_ln_call = pl.pallas_call(
    _ln_body,
    grid=(NP // BN,),
    in_specs=[
        pl.BlockSpec((T, BN, C), lambda nb: (0, nb, 0)),
        pl.BlockSpec((8, C), lambda nb: (0, 0)),
        pl.BlockSpec((8, C), lambda nb: (0, 0)),
        pl.BlockSpec((BN, T), lambda nb: (nb, 0)),
        pl.BlockSpec((BN, T), lambda nb: (nb, 0)),
    ],
    out_specs=pl.BlockSpec((T, BN, C), lambda nb: (0, nb, 0)),
    out_shape=jax.ShapeDtypeStruct((T, NP, C), jnp.float32),
)


def kernel(x, edge_index, edge_attr, W, b, ln_gamma, ln_beta):
    src = edge_index[0]
    dst = edge_index[1]
    pad = EP - E
    fill = (jnp.arange(pad, dtype=jnp.int32) * 37) % N
    src2 = jnp.concatenate([src, fill]).reshape(NG_TOT, G)
    dst2 = jnp.concatenate([dst, fill]).reshape(NG_TOT, G)
    ew2 = jnp.concatenate(
        [edge_attr, jnp.zeros((pad,), jnp.float32)]).reshape(NG_TOT, G)

    xT = jnp.transpose(x, (2, 0, 1))
    xflat = jnp.pad(xT, ((0, 0), (0, NP - N), (0, 0))).reshape(T * NP, C)

    zout, _, _ = _sc_aggregate(xflat, src2, dst2, ew2)

    y, s, s2 = _mm_call(zout, W, b.reshape(1, C))
    gp = jnp.pad(ln_gamma, ((0, NP - N), (0, 0)))
    bp = jnp.pad(ln_beta, ((0, NP - N), (0, 0)))
    outT = _ln_call(y, s, s2, gp, bp)

    return jnp.transpose(outT[:, :N, :], (1, 2, 0))
